# baseline (speedup 1.0000x reference)
import jax
import jax.numpy as jnp
import numpy as np
from jax import lax
from jax.experimental import pallas as pl
from jax.experimental.pallas import tpu as pltpu
from jax.experimental.pallas import tpu_sc as plsc

B = 16
N = 20000
C = 91
ROW = N * C
NSEL = 100
NB = 2048
SHIFT = 21
CHUNK = 14000
NCHUNK = ROW // CHUNK
STEPS = CHUNK // 16
CAP = 8192
PADK = 128
INT_MIN = np.int32(-2147483648)
INT_MAX = np.int32(2147483647)

_MESH = plsc.VectorSubcoreMesh(core_axis_name="c", subcore_axis_name="s")


def _lane():
    return lax.iota(jnp.int32, 16)


def _stage_a_body(logits_ref, candidx_ref, candval_ref, counts_ref,
                  buf0, buf1, hist, cidx, cvals, cnt16, sem0, sem1):
    wid = lax.axis_index("s") * 2 + lax.axis_index("c")

    @pl.when(wid < B)
    def _():
        b = wid
        lane = _lane()
        zeros16 = jnp.zeros((16,), jnp.int32)
        ones16 = jnp.ones((16,), jnp.int32)

        def _zero_hist(i, carry):
            hist[pl.ds(i * 16, 16)] = zeros16
            return carry
        lax.fori_loop(0, NB, _zero_hist, 0)

        def _zero_cand(i, carry):
            cidx[pl.ds(i * 16, 16)] = zeros16
            cvals[pl.ds(i * 16, 16)] = jnp.zeros((16,), jnp.float32)
            return carry
        lax.fori_loop(0, CAP // 16, _zero_cand, 0)

        def start(chunk, buf, sem):
            off = pl.multiple_of(b * ROW + chunk * CHUNK, 8)
            pltpu.async_copy(logits_ref.at[pl.ds(off, CHUNK)], buf, sem)

        def wait(buf, sem):
            pltpu.make_async_copy(
                logits_ref.at[pl.ds(0, CHUNK)], buf, sem).wait()

        def p1_compute(buf):
            def step(i, carry):
                v = buf[pl.ds(i * 16, 16)]
                u = lax.bitcast_convert_type(v, jnp.int32)
                m = u ^ (lax.shift_right_arithmetic(u, 31) | INT_MIN)
                addr = (lax.shift_left(lax.shift_right_logical(m, SHIFT), 4)
                        | lane)
                plsc.addupdate_scatter(hist, [addr], ones16)
                return carry
            lax.fori_loop(0, STEPS, step, 0, unroll=8)

        start(0, buf0, sem0)

        def p1_pair(g, carry):
            wait(buf0, sem0)
            start(2 * g + 1, buf1, sem1)
            p1_compute(buf0)
            wait(buf1, sem1)

            @pl.when(g < NCHUNK // 2 - 1)
            def _s():
                start(2 * g + 2, buf0, sem0)
            p1_compute(buf1)
            return carry
        lax.fori_loop(0, NCHUNK // 2, p1_pair, 0)

        def blocksum(jv):
            def acc16(t, a):
                return a + hist[pl.ds((jv * 16 + t) * 16, 16)]
            return jnp.sum(lax.fori_loop(0, 16, acc16, zeros16))

        def cond(st):
            jv, acc = st
            return (acc < NSEL) & (jv > 0)

        def body(st):
            jv, acc = st
            jv = jv - 1
            return jv, acc + blocksum(jv)
        jv, acc = lax.while_loop(cond, body,
                                 (jnp.int32(NB // 16), jnp.int32(0)))

        def binsums(k, tvc):
            s = jnp.sum(hist[pl.ds((jv * 16 + k) * 16, 16)])
            return jnp.where(lane == k, s, tvc)
        tv = lax.fori_loop(0, 16, binsums, zeros16)
        above = acc - jnp.sum(tv)
        cum = jnp.cumsum(lax.rev(tv, (0,))) + above
        kstar = jnp.max(plsc.all_reduce_ffs(cum >= NSEL))
        bstar_v = jnp.full((16,), jv * 16 + 15 - kstar, jnp.int32)
        kv = lax.shift_left(bstar_v, SHIFT)
        bits = jnp.where(kv < 0, kv & INT_MAX, ~kv)
        tvf = lax.bitcast_convert_type(bits, jnp.float32)

        def p2_compute(buf, chunk, off):
            base = chunk * CHUNK

            def step(i, off):
                v = buf[pl.ds(i * 16, 16)]
                msk = v >= tvf
                gi = (base + i * 16) + lane
                plsc.store_compressed(cidx.at[pl.ds(off, 16)], gi, mask=msk)
                plsc.store_compressed(cvals.at[pl.ds(off, 16)], v, mask=msk)
                cnt = jnp.sum(msk.astype(jnp.int32))
                return jnp.minimum(off + cnt, CAP - 16)
            return lax.fori_loop(0, STEPS, step, off, unroll=8)

        start(0, buf0, sem0)

        def p2_pair(g, off):
            wait(buf0, sem0)
            start(2 * g + 1, buf1, sem1)
            off = p2_compute(buf0, 2 * g, off)
            wait(buf1, sem1)

            @pl.when(g < NCHUNK // 2 - 1)
            def _s():
                start(2 * g + 2, buf0, sem0)
            off = p2_compute(buf1, 2 * g + 1, off)
            return off
        n = lax.fori_loop(0, NCHUNK // 2, p2_pair, jnp.int32(0))

        cnt16[...] = jnp.where(lane == 0, n, 0)
        ocap = pl.multiple_of(b * CAP, 8)
        o16 = pl.multiple_of(b * 16, 8)
        pltpu.sync_copy(cidx, candidx_ref.at[pl.ds(ocap, CAP)])
        pltpu.sync_copy(cvals, candval_ref.at[pl.ds(ocap, CAP)])
        pltpu.sync_copy(cnt16, counts_ref.at[pl.ds(o16, 16)])


_stage_a = pl.kernel(
    _stage_a_body,
    out_type=(
        jax.ShapeDtypeStruct((B * CAP,), jnp.int32),
        jax.ShapeDtypeStruct((B * CAP,), jnp.float32),
        jax.ShapeDtypeStruct((B * 16,), jnp.int32),
    ),
    mesh=_MESH,
    compiler_params=pltpu.CompilerParams(needs_layout_passes=False),
    scratch_types=[
        pltpu.VMEM((CHUNK,), jnp.float32),
        pltpu.VMEM((CHUNK,), jnp.float32),
        pltpu.VMEM((NB * 16,), jnp.int32),
        pltpu.VMEM((CAP,), jnp.int32),
        pltpu.VMEM((CAP,), jnp.float32),
        pltpu.VMEM((16,), jnp.int32),
        pltpu.SemaphoreType.DMA,
        pltpu.SemaphoreType.DMA,
    ],
)


def _stage_b_body(probs_ref, candidx_ref, counts_ref, boxes_ref, ts_ref,
                  scores_ref, labels_ref, boxout_ref,
                  pv, iv, selp, seli, labl, gidx, gbox, obox,
                  cnt16, ts16, sem):
    wid = lax.axis_index("s") * 2 + lax.axis_index("c")

    @pl.when(wid < B)
    def _():
        b = wid
        lane = _lane()
        ocap = pl.multiple_of(b * CAP, 8)
        o16 = pl.multiple_of(b * 16, 8)
        pltpu.sync_copy(probs_ref.at[pl.ds(ocap, CAP)], pv)
        pltpu.sync_copy(candidx_ref.at[pl.ds(ocap, CAP)], iv)
        pltpu.sync_copy(counts_ref.at[pl.ds(o16, 16)], cnt16)
        pltpu.sync_copy(ts_ref.at[pl.ds(o16, 16)], ts16)
        n = cnt16[...][0]
        nv = lax.div(n + 15, jnp.int32(16))
        tsv = ts16[...]
        hh = tsv[0]
        ww = tsv[1]

        def _zero_sel(q, carry):
            selp[pl.ds(q * 16, 16)] = jnp.zeros((16,), jnp.float32)
            seli[pl.ds(q * 16, 16)] = jnp.zeros((16,), jnp.int32)
            return carry
        lax.fori_loop(0, PADK // 16, _zero_sel, 0)

        def outer(k, pc):
            pval, pidx = pc

            def inner(j, bc):
                bv, biv = bc
                v = pv[pl.ds(j * 16, 16)]
                ivec = iv[pl.ds(j * 16, 16)]
                valid = (j * 16 + lane) < n
                elig = valid & ((v < pval) | ((v == pval) & (ivec > pidx)))
                better = elig & ((v > bv) | ((v == bv) & (ivec < biv)))
                bv = jnp.where(better, v, bv)
                biv = jnp.where(better, ivec, biv)
                return bv, biv
            bv, biv = lax.fori_loop(
                0, nv, inner,
                (jnp.full((16,), -1.0, jnp.float32),
                 jnp.full((16,), INT_MAX, jnp.int32)))
            mval = jnp.max(bv)
            midx = jnp.min(jnp.where(bv == mval, biv, INT_MAX))
            kv16 = jnp.full((16,), k, jnp.int32)
            lane0 = lane == 0
            plsc.store_scatter(selp, [kv16], jnp.full((16,), mval), mask=lane0)
            plsc.store_scatter(seli, [kv16], jnp.full((16,), midx), mask=lane0)
            return mval, midx
        lax.fori_loop(0, NSEL, outer, (jnp.float32(2.0), jnp.int32(-1)))

        def dec(q, carry):
            si = seli[pl.ds(q * 16, 16)]
            labl[pl.ds(q * 16, 16)] = lax.rem(si, jnp.int32(C))
            fb = b * (N * 4) + lax.div(si, jnp.int32(C)) * 4
            gidx[pl.ds(0 * PADK + q * 16, 16)] = fb
            gidx[pl.ds(1 * PADK + q * 16, 16)] = fb + 1
            gidx[pl.ds(2 * PADK + q * 16, 16)] = fb + 2
            gidx[pl.ds(3 * PADK + q * 16, 16)] = fb + 3
            return carry
        lax.fori_loop(0, PADK // 16, dec, 0)

        pltpu.async_copy(boxes_ref.at[gidx], gbox, sem).wait()

        def bx(q, carry):
            cx = gbox[pl.ds(0 * PADK + q * 16, 16)]
            cy = gbox[pl.ds(1 * PADK + q * 16, 16)]
            w = gbox[pl.ds(2 * PADK + q * 16, 16)]
            h = gbox[pl.ds(3 * PADK + q * 16, 16)]
            x0 = (cx - 0.5 * w) * ww
            y0 = (cy - 0.5 * h) * hh
            x1 = (cx + 0.5 * w) * ww
            y1 = (cy + 0.5 * h) * hh
            pos = q * 64 + lane * 4
            plsc.store_scatter(obox, [pos], x0)
            plsc.store_scatter(obox, [pos + 1], y0)
            plsc.store_scatter(obox, [pos + 2], x1)
            plsc.store_scatter(obox, [pos + 3], y1)
            return carry
        lax.fori_loop(0, PADK // 16, bx, 0)

        ok = pl.multiple_of(b * PADK, 8)
        ok4 = pl.multiple_of(b * PADK * 4, 8)
        pltpu.sync_copy(selp, scores_ref.at[pl.ds(ok, PADK)])
        pltpu.sync_copy(labl, labels_ref.at[pl.ds(ok, PADK)])
        pltpu.sync_copy(obox, boxout_ref.at[pl.ds(ok4, PADK * 4)])


_stage_b = pl.kernel(
    _stage_b_body,
    out_type=(
        jax.ShapeDtypeStruct((B * PADK,), jnp.float32),
        jax.ShapeDtypeStruct((B * PADK,), jnp.int32),
        jax.ShapeDtypeStruct((B * PADK * 4,), jnp.float32),
    ),
    mesh=_MESH,
    compiler_params=pltpu.CompilerParams(needs_layout_passes=False),
    scratch_types=[
        pltpu.VMEM((CAP,), jnp.float32),
        pltpu.VMEM((CAP,), jnp.int32),
        pltpu.VMEM((PADK,), jnp.float32),
        pltpu.VMEM((PADK,), jnp.int32),
        pltpu.VMEM((PADK,), jnp.int32),
        pltpu.VMEM((PADK * 4,), jnp.int32),
        pltpu.VMEM((PADK * 4,), jnp.float32),
        pltpu.VMEM((PADK * 4,), jnp.float32),
        pltpu.VMEM((16,), jnp.int32),
        pltpu.VMEM((16,), jnp.float32),
        pltpu.SemaphoreType.DMA,
    ],
)


def kernel(pred_logits, pred_boxes, target_sizes):
    flat = pred_logits.reshape(B * ROW)
    cand_idx, cand_val, counts = _stage_a(flat)
    probs = jax.nn.sigmoid(cand_val)
    ts = target_sizes.astype(jnp.float32)
    ts16 = jnp.concatenate(
        [ts, jnp.zeros((B, 14), jnp.float32)], axis=1).reshape(B * 16)
    boxes_flat = pred_boxes.reshape(B * N * 4)
    scores, labels, boxes = _stage_b(
        probs, cand_idx, counts, boxes_flat, ts16)
    return (scores.reshape(B, PADK)[:, :NSEL],
            labels.reshape(B, PADK)[:, :NSEL],
            boxes.reshape(B, PADK, 4)[:, :NSEL, :])


# speedup vs baseline: 2.9391x; 2.9391x over previous
import jax
import jax.numpy as jnp
import numpy as np
from jax import lax
from jax.experimental import pallas as pl
from jax.experimental.pallas import tpu as pltpu
from jax.experimental.pallas import tpu_sc as plsc

B = 16
N = 20000
C = 91
ROW = N * C
NSEL = 100
NB = 2048
SHIFT = 21
RPC = 200
NCHUNK = N // RPC
CHUNK = RPC * C
CAP = 8192
PADK = 128
INT_MIN = np.int32(-2147483648)
INT_MAX = np.int32(2147483647)

_MESH = plsc.VectorSubcoreMesh(core_axis_name="c", subcore_axis_name="s")


def _lane():
    return lax.iota(jnp.int32, 16)


_OFFS = (0, 16, 32, 48, 64, 75)


def _stage_a_body(logits_ref, candidx_ref, candval_ref, counts_ref,
                  buf0, buf1, hist, cidx, cvals, cnt16, sem0, sem1):
    wid = lax.axis_index("s") * 2 + lax.axis_index("c")

    @pl.when(wid < B)
    def _():
        b = wid
        lane = _lane()
        zeros16 = jnp.zeros((16,), jnp.int32)
        ones16 = jnp.ones((16,), jnp.int32)
        lane5 = lane >= 5

        def _zero_hist(i, carry):
            hist[pl.ds(i * 16, 16)] = zeros16
            return carry
        lax.fori_loop(0, NB, _zero_hist, 0)

        def _zero_cand(i, carry):
            cidx[pl.ds(i * 16, 16)] = zeros16
            cvals[pl.ds(i * 16, 16)] = jnp.zeros((16,), jnp.float32)
            return carry
        lax.fori_loop(0, CAP // 16, _zero_cand, 0)

        def start(chunk, buf, sem):
            r0 = pl.multiple_of(chunk * RPC, 8)
            pltpu.async_copy(logits_ref.at[b, pl.ds(r0, RPC), :], buf, sem)

        def wait(buf, sem):
            pltpu.make_async_copy(
                logits_ref.at[0, pl.ds(0, RPC), :], buf, sem).wait()

        def row_slices(buf, r):
            return [buf[r, pl.ds(o, 16)] for o in _OFFS]

        def bin_addrs(vs):
            addrs = []
            for v in vs:
                u = lax.bitcast_convert_type(v, jnp.int32)
                a = (lax.shift_right_arithmetic(u, 17)
                     & jnp.int32(0x7FF0)) | lane
                addrs.append(a)
            return addrs

        def hist_row(vs):
            addrs = bin_addrs(vs)
            for k in range(5):
                plsc.addupdate_scatter(hist, [addrs[k]], ones16)
            plsc.addupdate_scatter(hist, [addrs[5]], ones16, mask=lane5)

        def hist_chunk(buf):
            def row(r, carry):
                hist_row(row_slices(buf, r))
                return carry
            lax.fori_loop(0, RPC, row, 0, unroll=2)

        def append_row(vs, gbase, off, tvf):
            ms = [v >= tvf for v in vs]
            ms[5] = ms[5] & lane5
            cntv = zeros16
            for m in ms:
                cntv = cntv + m.astype(jnp.int32)
            total = jnp.sum(cntv)

            @pl.when(total > 0)
            def _slow():
                o = off
                for k in range(6):
                    gi = (gbase + _OFFS[k]) + lane
                    plsc.store_compressed(cidx.at[pl.ds(o, 16)], gi,
                                          mask=ms[k])
                    plsc.store_compressed(cvals.at[pl.ds(o, 16)], vs[k],
                                          mask=ms[k])
                    c = jnp.sum(ms[k].astype(jnp.int32))
                    o = jnp.minimum(o + c, CAP - 16)
            return jnp.minimum(off + total, CAP - 16)

        def append_chunk(buf, chunk, off, tvf):
            def row(r, off):
                vs = row_slices(buf, r)
                gbase = (chunk * RPC + r) * C
                return append_row(vs, gbase, off, tvf)
            return lax.fori_loop(0, RPC, row, off)

        def fused_chunk(buf, chunk, off, tvf):
            def row(r, off):
                vs = row_slices(buf, r)
                hist_row(vs)
                gbase = (chunk * RPC + r) * C
                return append_row(vs, gbase, off, tvf)
            return lax.fori_loop(0, RPC, row, off, unroll=2)

        def blocksum(jv):
            def acc16(t, a):
                return a + hist[pl.ds((jv * 16 + t) * 16, 16)]
            return jnp.sum(lax.fori_loop(0, 16, acc16, zeros16))

        def binsum_vec(jv):
            def binsums(k, tvc):
                s = jnp.sum(hist[pl.ds((jv * 16 + k) * 16, 16)])
                return jnp.where(lane == k, s, tvc)
            return lax.fori_loop(0, 16, binsums, zeros16)

        def scan_threshold():
            def cond_p(st):
                jv, acc = st
                return (acc < NSEL) & (jv > 0)

            def body_p(st):
                jv, acc = st
                jv = jv - 1
                return jv, acc + blocksum(jv)
            jp, accp = lax.while_loop(cond_p, body_p,
                                      (jnp.int32(64), jnp.int32(0)))

            def fine_pos(_):
                tv = binsum_vec(jp)
                above = accp - jnp.sum(tv)
                cum = jnp.cumsum(lax.rev(tv, (0,))) + above
                kstar = jnp.max(plsc.all_reduce_ffs(cum >= NSEL))
                binv = jnp.full((16,), jp * 16 + 15 - kstar, jnp.int32)
                return lax.bitcast_convert_type(
                    lax.shift_left(binv, SHIFT), jnp.float32)

            def phase_n(_):
                def cond_n(st):
                    jn, acc = st
                    return (acc < NSEL) & (jn < NB // 16)

                def body_n(st):
                    jn, acc = st
                    return jn + 1, acc + blocksum(jn)
                jn, accn = lax.while_loop(cond_n, body_n,
                                          (jnp.int32(64), accp))
                jb = jn - 1
                tv = binsum_vec(jb)
                above = accn - jnp.sum(tv)
                cum = jnp.cumsum(tv) + above
                kstar = jnp.max(plsc.all_reduce_ffs(cum >= NSEL))
                binv = jnp.full((16,), jb * 16 + kstar, jnp.int32)
                return lax.bitcast_convert_type(
                    lax.shift_left(binv, SHIFT) | jnp.int32(0x1FFFFF),
                    jnp.float32)
            return lax.cond(accp >= NSEL, fine_pos, phase_n, 0)

        start(0, buf0, sem0)
        wait(buf0, sem0)
        start(1, buf1, sem1)
        hist_chunk(buf0)
        tvf0 = scan_threshold()
        off0 = append_chunk(buf0, 0, jnp.int32(0), tvf0)

        def pair(g, carry):
            off, tvf = carry
            wait(buf1, sem1)
            start(2 * g + 2, buf0, sem0)
            off = fused_chunk(buf1, 2 * g + 1, off, tvf)
            wait(buf0, sem0)
            start(2 * g + 3, buf1, sem1)
            off = fused_chunk(buf0, 2 * g + 2, off, tvf)
            tvf = lax.cond(lax.rem(g, jnp.int32(4)) == 3,
                           lambda _: scan_threshold(), lambda _: tvf, 0)
            return off, tvf
        off, tvf = lax.fori_loop(0, (NCHUNK - 2) // 2, pair, (off0, tvf0))
        wait(buf1, sem1)
        n = fused_chunk(buf1, NCHUNK - 1, off, tvf)

        cnt16[...] = jnp.where(lane == 0, n, 0)
        ocap = pl.multiple_of(b * CAP, 8)
        o16 = pl.multiple_of(b * 16, 8)
        pltpu.sync_copy(cidx, candidx_ref.at[pl.ds(ocap, CAP)])
        pltpu.sync_copy(cvals, candval_ref.at[pl.ds(ocap, CAP)])
        pltpu.sync_copy(cnt16, counts_ref.at[pl.ds(o16, 16)])


_stage_a = pl.kernel(
    _stage_a_body,
    out_type=(
        jax.ShapeDtypeStruct((B * CAP,), jnp.int32),
        jax.ShapeDtypeStruct((B * CAP,), jnp.float32),
        jax.ShapeDtypeStruct((B * 16,), jnp.int32),
    ),
    mesh=_MESH,
    compiler_params=pltpu.CompilerParams(needs_layout_passes=False),
    scratch_types=[
        pltpu.VMEM((RPC, C), jnp.float32),
        pltpu.VMEM((RPC, C), jnp.float32),
        pltpu.VMEM((NB * 16,), jnp.int32),
        pltpu.VMEM((CAP,), jnp.int32),
        pltpu.VMEM((CAP,), jnp.float32),
        pltpu.VMEM((16,), jnp.int32),
        pltpu.SemaphoreType.DMA,
        pltpu.SemaphoreType.DMA,
    ],
)


def _stage_b_body(probs_ref, candidx_ref, counts_ref, boxes_ref, ts_ref,
                  scores_ref, labels_ref, boxout_ref,
                  pv, iv, selp, seli, labl, gidx, gbox, obox,
                  cnt16, ts16, sem):
    wid = lax.axis_index("s") * 2 + lax.axis_index("c")

    @pl.when(wid < B)
    def _():
        b = wid
        lane = _lane()
        ocap = pl.multiple_of(b * CAP, 8)
        o16 = pl.multiple_of(b * 16, 8)
        pltpu.sync_copy(probs_ref.at[pl.ds(ocap, CAP)], pv)
        pltpu.sync_copy(candidx_ref.at[pl.ds(ocap, CAP)], iv)
        pltpu.sync_copy(counts_ref.at[pl.ds(o16, 16)], cnt16)
        pltpu.sync_copy(ts_ref.at[pl.ds(o16, 16)], ts16)
        n = cnt16[...][0]
        nv = lax.div(n + 15, jnp.int32(16))
        tsv = ts16[...]
        hh = tsv[0]
        ww = tsv[1]

        def _zero_sel(q, carry):
            selp[pl.ds(q * 16, 16)] = jnp.zeros((16,), jnp.float32)
            seli[pl.ds(q * 16, 16)] = jnp.zeros((16,), jnp.int32)
            return carry
        lax.fori_loop(0, PADK // 16, _zero_sel, 0)

        def outer(k, pc):
            pval, pidx = pc

            def inner(j, bc):
                bv, biv = bc
                v = pv[pl.ds(j * 16, 16)]
                ivec = iv[pl.ds(j * 16, 16)]
                valid = (j * 16 + lane) < n
                elig = valid & ((v < pval) | ((v == pval) & (ivec > pidx)))
                better = elig & ((v > bv) | ((v == bv) & (ivec < biv)))
                bv = jnp.where(better, v, bv)
                biv = jnp.where(better, ivec, biv)
                return bv, biv
            bv, biv = lax.fori_loop(
                0, nv, inner,
                (jnp.full((16,), -1.0, jnp.float32),
                 jnp.full((16,), INT_MAX, jnp.int32)))
            mval = jnp.max(bv)
            midx = jnp.min(jnp.where(bv == mval, biv, INT_MAX))
            kv16 = jnp.full((16,), k, jnp.int32)
            lane0 = lane == 0
            plsc.store_scatter(selp, [kv16], jnp.full((16,), mval), mask=lane0)
            plsc.store_scatter(seli, [kv16], jnp.full((16,), midx), mask=lane0)
            return mval, midx
        lax.fori_loop(0, NSEL, outer, (jnp.float32(2.0), jnp.int32(-1)))

        def dec(q, carry):
            si = seli[pl.ds(q * 16, 16)]
            labl[pl.ds(q * 16, 16)] = lax.rem(si, jnp.int32(C))
            fb = b * (N * 4) + lax.div(si, jnp.int32(C)) * 4
            gidx[pl.ds(0 * PADK + q * 16, 16)] = fb
            gidx[pl.ds(1 * PADK + q * 16, 16)] = fb + 1
            gidx[pl.ds(2 * PADK + q * 16, 16)] = fb + 2
            gidx[pl.ds(3 * PADK + q * 16, 16)] = fb + 3
            return carry
        lax.fori_loop(0, PADK // 16, dec, 0)

        pltpu.async_copy(boxes_ref.at[gidx], gbox, sem).wait()

        def bx(q, carry):
            cx = gbox[pl.ds(0 * PADK + q * 16, 16)]
            cy = gbox[pl.ds(1 * PADK + q * 16, 16)]
            w = gbox[pl.ds(2 * PADK + q * 16, 16)]
            h = gbox[pl.ds(3 * PADK + q * 16, 16)]
            x0 = (cx - 0.5 * w) * ww
            y0 = (cy - 0.5 * h) * hh
            x1 = (cx + 0.5 * w) * ww
            y1 = (cy + 0.5 * h) * hh
            pos = q * 64 + lane * 4
            plsc.store_scatter(obox, [pos], x0)
            plsc.store_scatter(obox, [pos + 1], y0)
            plsc.store_scatter(obox, [pos + 2], x1)
            plsc.store_scatter(obox, [pos + 3], y1)
            return carry
        lax.fori_loop(0, PADK // 16, bx, 0)

        ok = pl.multiple_of(b * PADK, 8)
        ok4 = pl.multiple_of(b * PADK * 4, 8)
        pltpu.sync_copy(selp, scores_ref.at[pl.ds(ok, PADK)])
        pltpu.sync_copy(labl, labels_ref.at[pl.ds(ok, PADK)])
        pltpu.sync_copy(obox, boxout_ref.at[pl.ds(ok4, PADK * 4)])


_stage_b = pl.kernel(
    _stage_b_body,
    out_type=(
        jax.ShapeDtypeStruct((B * PADK,), jnp.float32),
        jax.ShapeDtypeStruct((B * PADK,), jnp.int32),
        jax.ShapeDtypeStruct((B * PADK * 4,), jnp.float32),
    ),
    mesh=_MESH,
    compiler_params=pltpu.CompilerParams(needs_layout_passes=False),
    scratch_types=[
        pltpu.VMEM((CAP,), jnp.float32),
        pltpu.VMEM((CAP,), jnp.int32),
        pltpu.VMEM((PADK,), jnp.float32),
        pltpu.VMEM((PADK,), jnp.int32),
        pltpu.VMEM((PADK,), jnp.int32),
        pltpu.VMEM((PADK * 4,), jnp.int32),
        pltpu.VMEM((PADK * 4,), jnp.float32),
        pltpu.VMEM((PADK * 4,), jnp.float32),
        pltpu.VMEM((16,), jnp.int32),
        pltpu.VMEM((16,), jnp.float32),
        pltpu.SemaphoreType.DMA,
    ],
)


def kernel(pred_logits, pred_boxes, target_sizes):
    cand_idx, cand_val, counts = _stage_a(pred_logits)
    probs = jax.nn.sigmoid(cand_val)
    ts = target_sizes.astype(jnp.float32)
    ts16 = jnp.concatenate(
        [ts, jnp.zeros((B, 14), jnp.float32)], axis=1).reshape(B * 16)
    boxes_flat = pred_boxes.reshape(B * N * 4)
    scores, labels, boxes = _stage_b(
        probs, cand_idx, counts, boxes_flat, ts16)
    return (scores.reshape(B, PADK)[:, :NSEL],
            labels.reshape(B, PADK)[:, :NSEL],
            boxes.reshape(B, PADK, 4)[:, :NSEL, :])


# speedup vs baseline: 4.5730x; 1.5559x over previous
import jax
import jax.numpy as jnp
import numpy as np
from jax import lax
from jax.experimental import pallas as pl
from jax.experimental.pallas import tpu as pltpu
from jax.experimental.pallas import tpu_sc as plsc

B = 16
N = 20000
C = 91
ROW = N * C
NSEL = 100
NB = 2048
SHIFT = 21
RPC = 200
NCHUNK = N // RPC
SEC = 20
CAP = 8192
PADK = 128
INT_MIN = np.int32(-2147483648)
INT_MAX = np.int32(2147483647)

_MESH = plsc.VectorSubcoreMesh(core_axis_name="c", subcore_axis_name="s")


def _lane():
    return lax.iota(jnp.int32, 16)


_OFFS = (0, 16, 32, 48, 64, 75)


def _stage_a_body(logits_ref, candidx_ref, candval_ref, counts_ref,
                  buf0, buf1, hist, cidx, cvals, cidx2, cvals2, ctr_ref,
                  cnt16, sem0, sem1):
    wid = lax.axis_index("s") * 2 + lax.axis_index("c")

    @pl.when(wid < B)
    def _():
        b = wid
        lane = _lane()
        zeros16 = jnp.zeros((16,), jnp.int32)
        ones16 = jnp.ones((16,), jnp.int32)
        lane5 = lane >= 5

        def _zero_hist(i, carry):
            hist[pl.ds(i * 16, 16)] = zeros16
            return carry
        lax.fori_loop(0, NB, _zero_hist, 0)

        def _zero_cand(i, carry):
            cidx2[pl.ds(i * 16, 16)] = zeros16
            cvals2[pl.ds(i * 16, 16)] = jnp.zeros((16,), jnp.float32)
            return carry
        lax.fori_loop(0, (CAP + 16) // 16, _zero_cand, 0)

        def start(chunk, buf, sem):
            r0 = pl.multiple_of(chunk * RPC, 8)
            pltpu.async_copy(logits_ref.at[b, pl.ds(r0, RPC), :], buf, sem)

        def wait(buf, sem):
            pltpu.make_async_copy(
                logits_ref.at[0, pl.ds(0, RPC), :], buf, sem).wait()

        def row_slices(buf, r):
            return [buf[r, pl.ds(o, 16)] for o in _OFFS]

        def bin_addrs(vs):
            addrs = []
            for v in vs:
                u = lax.bitcast_convert_type(v, jnp.int32)
                a = (lax.shift_right_arithmetic(u, 17)
                     & jnp.int32(0x7FF0)) | lane
                addrs.append(a)
            return addrs

        def hist_row(vs):
            addrs = bin_addrs(vs)
            for k in range(5):
                plsc.addupdate_scatter(hist, [addrs[k]], ones16)
            plsc.addupdate_scatter(hist, [addrs[5]], ones16, mask=lane5)

        def hist_chunk(buf):
            def row(r, carry):
                hist_row(row_slices(buf, r))
                return carry
            lax.fori_loop(0, RPC, row, 0, unroll=2)

        neginf = jnp.full((16,), -jnp.inf, jnp.float32)
        laneoff = lax.shift_left(lane, 9)

        def row_max(vs):
            m = jnp.maximum(vs[0], vs[1])
            m = jnp.maximum(m, vs[2])
            m = jnp.maximum(m, vs[3])
            m = jnp.maximum(m, vs[4])
            return jnp.maximum(m, jnp.where(lane5, vs[5], neginf))

        def slow_section(buf, chunk, s0, tvf):
            def row(r, ctrv):
                vs = row_slices(buf, s0 + r)
                gbase = (chunk * RPC + s0 + r) * C
                for k in range(6):
                    m = vs[k] >= tvf
                    if k == 5:
                        m = m & lane5
                    addr = laneoff + ctrv
                    gi = (gbase + _OFFS[k]) + lane
                    plsc.store_scatter(cidx, [addr], gi, mask=m)
                    plsc.store_scatter(cvals, [addr], vs[k], mask=m)
                    ctrv = jnp.minimum(ctrv + m.astype(jnp.int32), 511)
                return ctrv
            ctrv = lax.fori_loop(0, SEC, row, ctr_ref[...])
            ctr_ref[...] = ctrv

        def proc_chunk(buf, chunk, tvf, do_hist):
            def sec(s, carry):
                s0 = s * SEC

                def hotrow(i, acc):
                    vs = row_slices(buf, s0 + i)
                    if do_hist:
                        hist_row(vs)
                    return jnp.maximum(acc, row_max(vs))
                acc = lax.fori_loop(0, SEC, hotrow, neginf, unroll=2)
                hit = jnp.sum((acc >= tvf).astype(jnp.int32))

                @pl.when(hit > 0)
                def _():
                    slow_section(buf, chunk, s0, tvf)
                return carry
            lax.fori_loop(0, RPC // SEC, sec, 0)

        def blocksum(jv):
            def acc16(t, a):
                return a + hist[pl.ds((jv * 16 + t) * 16, 16)]
            return jnp.sum(lax.fori_loop(0, 16, acc16, zeros16))

        def binsum_vec(jv):
            def binsums(k, tvc):
                s = jnp.sum(hist[pl.ds((jv * 16 + k) * 16, 16)])
                return jnp.where(lane == k, s, tvc)
            return lax.fori_loop(0, 16, binsums, zeros16)

        def scan_threshold():
            def cond_p(st):
                jv, acc = st
                return (acc < NSEL) & (jv > 0)

            def body_p(st):
                jv, acc = st
                jv = jv - 1
                return jv, acc + blocksum(jv)
            jp, accp = lax.while_loop(cond_p, body_p,
                                      (jnp.int32(64), jnp.int32(0)))

            def fine_pos(_):
                tv = binsum_vec(jp)
                above = accp - jnp.sum(tv)
                cum = jnp.cumsum(lax.rev(tv, (0,))) + above
                kstar = jnp.max(plsc.all_reduce_ffs(cum >= NSEL))
                binv = jnp.full((16,), jp * 16 + 15 - kstar, jnp.int32)
                return lax.bitcast_convert_type(
                    lax.shift_left(binv, SHIFT), jnp.float32)

            def phase_n(_):
                def cond_n(st):
                    jn, acc = st
                    return (acc < NSEL) & (jn < NB // 16)

                def body_n(st):
                    jn, acc = st
                    return jn + 1, acc + blocksum(jn)
                jn, accn = lax.while_loop(cond_n, body_n,
                                          (jnp.int32(64), accp))
                jb = jn - 1
                tv = binsum_vec(jb)
                above = accn - jnp.sum(tv)
                cum = jnp.cumsum(tv) + above
                kstar = jnp.max(plsc.all_reduce_ffs(cum >= NSEL))
                binv = jnp.full((16,), jb * 16 + kstar, jnp.int32)
                return lax.bitcast_convert_type(
                    lax.shift_left(binv, SHIFT) | jnp.int32(0x1FFFFF),
                    jnp.float32)
            return lax.cond(accp >= NSEL, fine_pos, phase_n, 0)

        ctr_ref[...] = zeros16
        start(0, buf0, sem0)
        wait(buf0, sem0)
        start(1, buf1, sem1)
        hist_chunk(buf0)
        tvf0 = scan_threshold()
        proc_chunk(buf0, jnp.int32(0), tvf0, do_hist=False)

        def pair(g, tvf):
            wait(buf1, sem1)
            start(2 * g + 2, buf0, sem0)
            proc_chunk(buf1, 2 * g + 1, tvf, do_hist=True)
            wait(buf0, sem0)
            start(2 * g + 3, buf1, sem1)
            proc_chunk(buf0, 2 * g + 2, tvf, do_hist=True)
            rescan = (g < 2) | (lax.rem(g, jnp.int32(4)) == 3)
            return lax.cond(rescan, lambda _: scan_threshold(),
                            lambda _: tvf, 0)
        tvf = lax.fori_loop(0, (NCHUNK - 2) // 2, pair, tvf0)
        wait(buf1, sem1)
        proc_chunk(buf1, jnp.int32(NCHUNK - 1), tvf, do_hist=True)

        ctrv = ctr_ref[...]
        cs = jnp.cumsum(ctrv)
        n = jnp.max(cs)
        excl = cs - ctrv
        for el in range(16):
            cnt_l = ctrv[el]
            dst0 = excl[el]

            def mv(i, carry):
                srci = cidx[pl.ds(el * 512 + i * 16, 16)]
                srcv = cvals[pl.ds(el * 512 + i * 16, 16)]
                m = (i * 16 + lane) < cnt_l
                cidx2[pl.ds(dst0 + i * 16, 16)] = jnp.where(m, srci, 0)
                cvals2[pl.ds(dst0 + i * 16, 16)] = jnp.where(
                    m, srcv, jnp.float32(0.0))
                return carry
            lax.fori_loop(0, lax.div(cnt_l + 15, jnp.int32(16)), mv, 0)

        cnt16[...] = jnp.where(lane == 0, n, 0)
        ocap = pl.multiple_of(b * CAP, 8)
        o16 = pl.multiple_of(b * 16, 8)
        pltpu.sync_copy(cidx2.at[pl.ds(0, CAP)],
                        candidx_ref.at[pl.ds(ocap, CAP)])
        pltpu.sync_copy(cvals2.at[pl.ds(0, CAP)],
                        candval_ref.at[pl.ds(ocap, CAP)])
        pltpu.sync_copy(cnt16, counts_ref.at[pl.ds(o16, 16)])


_stage_a = pl.kernel(
    _stage_a_body,
    out_type=(
        jax.ShapeDtypeStruct((B * CAP,), jnp.int32),
        jax.ShapeDtypeStruct((B * CAP,), jnp.float32),
        jax.ShapeDtypeStruct((B * 16,), jnp.int32),
    ),
    mesh=_MESH,
    compiler_params=pltpu.CompilerParams(needs_layout_passes=False),
    scratch_types=[
        pltpu.VMEM((RPC, C), jnp.float32),
        pltpu.VMEM((RPC, C), jnp.float32),
        pltpu.VMEM((NB * 16,), jnp.int32),
        pltpu.VMEM((CAP,), jnp.int32),
        pltpu.VMEM((CAP,), jnp.float32),
        pltpu.VMEM((CAP + 16,), jnp.int32),
        pltpu.VMEM((CAP + 16,), jnp.float32),
        pltpu.VMEM((16,), jnp.int32),
        pltpu.VMEM((16,), jnp.int32),
        pltpu.SemaphoreType.DMA,
        pltpu.SemaphoreType.DMA,
    ],
)


def _stage_b_body(probs_ref, candidx_ref, counts_ref, boxes_ref, ts_ref,
                  scores_ref, labels_ref, boxout_ref,
                  pv, iv, selp, seli, labl, gidx, gbox, obox,
                  cnt16, ts16, sem):
    wid = lax.axis_index("s") * 2 + lax.axis_index("c")

    @pl.when(wid < B)
    def _():
        b = wid
        lane = _lane()
        ocap = pl.multiple_of(b * CAP, 8)
        o16 = pl.multiple_of(b * 16, 8)
        pltpu.sync_copy(probs_ref.at[pl.ds(ocap, CAP)], pv)
        pltpu.sync_copy(candidx_ref.at[pl.ds(ocap, CAP)], iv)
        pltpu.sync_copy(counts_ref.at[pl.ds(o16, 16)], cnt16)
        pltpu.sync_copy(ts_ref.at[pl.ds(o16, 16)], ts16)
        n = cnt16[...][0]
        nv = lax.div(n + 15, jnp.int32(16))
        tsv = ts16[...]
        hh = tsv[0]
        ww = tsv[1]

        def _zero_sel(q, carry):
            selp[pl.ds(q * 16, 16)] = jnp.zeros((16,), jnp.float32)
            seli[pl.ds(q * 16, 16)] = jnp.zeros((16,), jnp.int32)
            return carry
        lax.fori_loop(0, PADK // 16, _zero_sel, 0)

        def outer(k, pc):
            pval, pidx = pc

            def inner(j, bc):
                bv, biv = bc
                v = pv[pl.ds(j * 16, 16)]
                ivec = iv[pl.ds(j * 16, 16)]
                valid = (j * 16 + lane) < n
                elig = valid & ((v < pval) | ((v == pval) & (ivec > pidx)))
                better = elig & ((v > bv) | ((v == bv) & (ivec < biv)))
                bv = jnp.where(better, v, bv)
                biv = jnp.where(better, ivec, biv)
                return bv, biv
            bv, biv = lax.fori_loop(
                0, nv, inner,
                (jnp.full((16,), -1.0, jnp.float32),
                 jnp.full((16,), INT_MAX, jnp.int32)))
            mval = jnp.max(bv)
            midx = jnp.min(jnp.where(bv == mval, biv, INT_MAX))
            kv16 = jnp.full((16,), k, jnp.int32)
            lane0 = lane == 0
            plsc.store_scatter(selp, [kv16], jnp.full((16,), mval), mask=lane0)
            plsc.store_scatter(seli, [kv16], jnp.full((16,), midx), mask=lane0)
            return mval, midx
        lax.fori_loop(0, NSEL, outer, (jnp.float32(2.0), jnp.int32(-1)))

        def dec(q, carry):
            si = seli[pl.ds(q * 16, 16)]
            labl[pl.ds(q * 16, 16)] = lax.rem(si, jnp.int32(C))
            fb = b * (N * 4) + lax.div(si, jnp.int32(C)) * 4
            gidx[pl.ds(0 * PADK + q * 16, 16)] = fb
            gidx[pl.ds(1 * PADK + q * 16, 16)] = fb + 1
            gidx[pl.ds(2 * PADK + q * 16, 16)] = fb + 2
            gidx[pl.ds(3 * PADK + q * 16, 16)] = fb + 3
            return carry
        lax.fori_loop(0, PADK // 16, dec, 0)

        pltpu.async_copy(boxes_ref.at[gidx], gbox, sem).wait()

        def bx(q, carry):
            cx = gbox[pl.ds(0 * PADK + q * 16, 16)]
            cy = gbox[pl.ds(1 * PADK + q * 16, 16)]
            w = gbox[pl.ds(2 * PADK + q * 16, 16)]
            h = gbox[pl.ds(3 * PADK + q * 16, 16)]
            x0 = (cx - 0.5 * w) * ww
            y0 = (cy - 0.5 * h) * hh
            x1 = (cx + 0.5 * w) * ww
            y1 = (cy + 0.5 * h) * hh
            pos = q * 64 + lane * 4
            plsc.store_scatter(obox, [pos], x0)
            plsc.store_scatter(obox, [pos + 1], y0)
            plsc.store_scatter(obox, [pos + 2], x1)
            plsc.store_scatter(obox, [pos + 3], y1)
            return carry
        lax.fori_loop(0, PADK // 16, bx, 0)

        ok = pl.multiple_of(b * PADK, 8)
        ok4 = pl.multiple_of(b * PADK * 4, 8)
        pltpu.sync_copy(selp, scores_ref.at[pl.ds(ok, PADK)])
        pltpu.sync_copy(labl, labels_ref.at[pl.ds(ok, PADK)])
        pltpu.sync_copy(obox, boxout_ref.at[pl.ds(ok4, PADK * 4)])


_stage_b = pl.kernel(
    _stage_b_body,
    out_type=(
        jax.ShapeDtypeStruct((B * PADK,), jnp.float32),
        jax.ShapeDtypeStruct((B * PADK,), jnp.int32),
        jax.ShapeDtypeStruct((B * PADK * 4,), jnp.float32),
    ),
    mesh=_MESH,
    compiler_params=pltpu.CompilerParams(needs_layout_passes=False),
    scratch_types=[
        pltpu.VMEM((CAP,), jnp.float32),
        pltpu.VMEM((CAP,), jnp.int32),
        pltpu.VMEM((PADK,), jnp.float32),
        pltpu.VMEM((PADK,), jnp.int32),
        pltpu.VMEM((PADK,), jnp.int32),
        pltpu.VMEM((PADK * 4,), jnp.int32),
        pltpu.VMEM((PADK * 4,), jnp.float32),
        pltpu.VMEM((PADK * 4,), jnp.float32),
        pltpu.VMEM((16,), jnp.int32),
        pltpu.VMEM((16,), jnp.float32),
        pltpu.SemaphoreType.DMA,
    ],
)


def kernel(pred_logits, pred_boxes, target_sizes):
    cand_idx, cand_val, counts = _stage_a(pred_logits)
    probs = jax.nn.sigmoid(cand_val)
    ts = target_sizes.astype(jnp.float32)
    ts16 = jnp.concatenate(
        [ts, jnp.zeros((B, 14), jnp.float32)], axis=1).reshape(B * 16)
    boxes_flat = pred_boxes.reshape(B * N * 4)
    scores, labels, boxes = _stage_b(
        probs, cand_idx, counts, boxes_flat, ts16)
    return (scores.reshape(B, PADK)[:, :NSEL],
            labels.reshape(B, PADK)[:, :NSEL],
            boxes.reshape(B, PADK, 4)[:, :NSEL, :])
